# Initial kernel scaffold; baseline (speedup 1.0000x reference)
import jax
import jax.numpy as jnp
from jax.experimental import pallas as pl


def kernel(x, tables, W, b):
    raise NotImplementedError("write your pallas kernel here")


# speedup vs baseline: 1.8950x; 1.8950x over previous
import functools

import jax
import jax.numpy as jnp
from jax import lax
from jax.experimental import pallas as pl
from jax.experimental.pallas import tpu as pltpu
from jax.experimental.pallas import tpu_sc as plsc

NUM_ATTR = 26
VOCAB = 1000
D = 128
BATCH = 1024
OBJ = 20
FIN = NUM_ATTR * D
N = BATCH * OBJ

NC = 2
NS = 16
NW = NC * NS
ROWS_W = N // NW
CH = 128
NCH = ROWS_W // CH

NP = NUM_ATTR // 2
BM = 256


def _gather(idx, tab):
    mesh = plsc.VectorSubcoreMesh(core_axis_name="c", subcore_axis_name="s")

    @functools.partial(
        pl.kernel,
        out_type=jax.ShapeDtypeStruct((NUM_ATTR, N, D), jnp.float32),
        mesh=mesh,
        scratch_types=[
            pltpu.VMEM((NCH, CH), jnp.int32),
            pltpu.VMEM((ROWS_W, D), jnp.float32),
            pltpu.SemaphoreType.DMA,
        ],
    )
    def sc_kernel(idx_ref, tab_ref, out_ref, idx_v, rows_v, sem):
        wid = lax.axis_index("s") * NC + lax.axis_index("c")
        base = wid * ROWS_W

        def body(a, carry):
            pltpu.sync_copy(idx_ref.at[wid, a], idx_v)
            cps = [
                pltpu.async_copy(tab_ref.at[idx_v.at[j]],
                                 rows_v.at[pl.ds(j * CH, CH)], sem)
                for j in range(NCH)
            ]
            for c in cps:
                c.wait()
            pltpu.sync_copy(rows_v, out_ref.at[a, pl.ds(base, ROWS_W)])
            return carry

        lax.fori_loop(0, NUM_ATTR, body, 0)

    return sc_kernel(idx, tab)


def _matmul(g, wt, bias):
    def body(g_ref, w_ref, b_ref, o_ref):
        acc = jnp.broadcast_to(b_ref[...], (BM, FIN)).astype(jnp.float32)
        for p in range(NP):
            a = jnp.concatenate([g_ref[2 * p], g_ref[2 * p + 1]], axis=-1)
            acc = acc + jnp.dot(a.astype(jnp.bfloat16), w_ref[p],
                                preferred_element_type=jnp.float32)
        o_ref[...] = acc

    return pl.pallas_call(
        body,
        grid=(N // BM,),
        in_specs=[
            pl.BlockSpec((NUM_ATTR, BM, D), lambda i: (0, i, 0)),
            pl.BlockSpec((NP, 2 * D, FIN), lambda i: (0, 0, 0)),
            pl.BlockSpec((1, FIN), lambda i: (0, 0)),
        ],
        out_specs=pl.BlockSpec((BM, FIN), lambda i: (i, 0)),
        out_shape=jax.ShapeDtypeStruct((N, FIN), jnp.float32),
    )(g, wt, bias)


def kernel(x, tables, W, b):
    xt = x.reshape(N, NUM_ATTR).T
    idx = xt + jnp.arange(NUM_ATTR, dtype=jnp.int32)[:, None] * VOCAB
    idx = idx.reshape(NUM_ATTR, NW, NCH, CH).transpose(1, 0, 2, 3)
    tab = tables.reshape(NUM_ATTR * VOCAB, D)
    g = _gather(idx, tab)
    wt = W.T.reshape(NP, 2 * D, FIN).astype(jnp.bfloat16)
    y = _matmul(g, wt, b.reshape(1, FIN))
    return y.reshape(BATCH, OBJ, FIN)


# speedup vs baseline: 3.1474x; 1.6609x over previous
import functools

import jax
import jax.numpy as jnp
from jax import lax
from jax.experimental import pallas as pl
from jax.experimental.pallas import tpu as pltpu
from jax.experimental.pallas import tpu_sc as plsc

NUM_ATTR = 26
VOCAB = 1000
D = 128
DW = D // 2
BATCH = 1024
OBJ = 20
FIN = NUM_ATTR * D
N = BATCH * OBJ

NC = 2
NS = 16
NW = NC * NS
ROWS_W = N // NW
CH = 128
NCH = ROWS_W // CH

NP = NUM_ATTR // 2
BB = 16
BM = BB * OBJ


def _gather(idx, tab):
    mesh = plsc.VectorSubcoreMesh(core_axis_name="c", subcore_axis_name="s")

    @functools.partial(
        pl.kernel,
        out_type=jax.ShapeDtypeStruct((NUM_ATTR, N, D), jnp.float32),
        mesh=mesh,
        scratch_types=[
            pltpu.VMEM((NCH, CH), jnp.int32),
            pltpu.VMEM((ROWS_W, D), jnp.float32),
            pltpu.SemaphoreType.DMA,
        ],
    )
    def sc_kernel(idx_ref, tab_ref, out_ref, idx_v, rows_v, sem):
        wid = lax.axis_index("s") * NC + lax.axis_index("c")
        base = wid * ROWS_W

        def body(a, carry):
            pltpu.sync_copy(idx_ref.at[wid, a], idx_v)
            cps = [
                pltpu.async_copy(tab_ref.at[idx_v.at[j]],
                                 rows_v.at[pl.ds(j * CH, CH)], sem)
                for j in range(NCH)
            ]
            for c in cps:
                c.wait()
            pltpu.sync_copy(rows_v, out_ref.at[a, pl.ds(base, ROWS_W)])
            return carry

        lax.fori_loop(0, NUM_ATTR, body, 0)

    return sc_kernel(idx, tab)


def _cast_bf16(w):
    def body(w_ref, o_ref):
        o_ref[...] = w_ref[...].astype(jnp.bfloat16)

    return pl.pallas_call(
        body,
        grid=(NUM_ATTR,),
        in_specs=[pl.BlockSpec((D, FIN), lambda i: (i, 0))],
        out_specs=pl.BlockSpec((D, FIN), lambda i: (i, 0)),
        out_shape=jax.ShapeDtypeStruct((FIN, FIN), jnp.bfloat16),
    )(w)


def _matmul(g, wb, bias):
    dn = (((1,), (1,)), ((), ()))

    def body(g_ref, w_ref, b_ref, o_ref):
        acc = jnp.broadcast_to(b_ref[...], (BM, FIN)).astype(jnp.float32)
        for p in range(NP):
            a = jnp.concatenate([g_ref[2 * p], g_ref[2 * p + 1]], axis=-1)
            acc = acc + lax.dot_general(
                a.astype(jnp.bfloat16), w_ref[:, 2 * D * p:2 * D * (p + 1)],
                dn, preferred_element_type=jnp.float32)
        o_ref[...] = acc

    return pl.pallas_call(
        body,
        grid=(N // BM,),
        in_specs=[
            pl.BlockSpec((NUM_ATTR, BM, D), lambda i: (0, i, 0)),
            pl.BlockSpec((FIN, FIN), lambda i: (0, 0)),
            pl.BlockSpec((1, FIN), lambda i: (0, 0)),
        ],
        out_specs=pl.BlockSpec((BM, FIN), lambda i: (i, 0)),
        out_shape=jax.ShapeDtypeStruct((N, FIN), jnp.float32),
    )(g, wb, bias)


def kernel(x, tables, W, b):
    xt = x.transpose(2, 1, 0).reshape(NUM_ATTR, N)
    idx = xt + jnp.arange(NUM_ATTR, dtype=jnp.int32)[:, None] * VOCAB
    idx = idx.reshape(NUM_ATTR, NW, NCH, CH).transpose(1, 0, 2, 3)
    tab = tables.reshape(NUM_ATTR * VOCAB, D)
    g = _gather(idx, tab)
    wb = _cast_bf16(W)
    y = _matmul(g, wb, b.reshape(1, FIN))
    return y.reshape(OBJ, BATCH, FIN).transpose(1, 0, 2)


# speedup vs baseline: 3.6525x; 1.1605x over previous
import functools

import jax
import jax.numpy as jnp
from jax import lax
from jax.experimental import pallas as pl
from jax.experimental.pallas import tpu as pltpu
from jax.experimental.pallas import tpu_sc as plsc

NUM_ATTR = 26
VOCAB = 1000
D = 128
BATCH = 1024
OBJ = 20
FIN = NUM_ATTR * D
N = BATCH * OBJ

NC = 2
NS = 16
NW = NC * NS
CH = 128

CHUNKS = 5
ROWS_C = N // CHUNKS
ROWS_WC = ROWS_C // NW
NCH = ROWS_WC // CH

NP = NUM_ATTR // 2
BM = 256
CBLK = ROWS_C // BM


def _gather(idx, tab):
    mesh = plsc.VectorSubcoreMesh(core_axis_name="c", subcore_axis_name="s")

    @functools.partial(
        pl.kernel,
        out_type=jax.ShapeDtypeStruct((NUM_ATTR, ROWS_C, D), jnp.float32),
        mesh=mesh,
        scratch_types=[
            pltpu.VMEM((NCH, CH), jnp.int32),
            pltpu.VMEM((ROWS_WC, D), jnp.float32),
            pltpu.SemaphoreType.DMA,
        ],
    )
    def sc_kernel(idx_ref, tab_ref, out_ref, idx_v, rows_v, sem):
        wid = lax.axis_index("s") * NC + lax.axis_index("c")
        base = wid * ROWS_WC

        def body(a, carry):
            pltpu.sync_copy(idx_ref.at[wid, a], idx_v)
            cps = [
                pltpu.async_copy(tab_ref.at[idx_v.at[j]],
                                 rows_v.at[pl.ds(j * CH, CH)], sem)
                for j in range(NCH)
            ]
            for c in cps:
                c.wait()
            pltpu.sync_copy(rows_v, out_ref.at[a, pl.ds(base, ROWS_WC)])
            return carry

        lax.fori_loop(0, NUM_ATTR, body, 0)

    return sc_kernel(idx, tab)


def _cast_bf16(w):
    def body(w_ref, o_ref):
        o_ref[...] = w_ref[...].astype(jnp.bfloat16)

    return pl.pallas_call(
        body,
        grid=(NUM_ATTR,),
        in_specs=[pl.BlockSpec((D, FIN), lambda i: (i, 0))],
        out_specs=pl.BlockSpec((D, FIN), lambda i: (i, 0)),
        out_shape=jax.ShapeDtypeStruct((FIN, FIN), jnp.bfloat16),
    )(w)


def _mm_body(g_ref, w_ref, b_ref, o_ref):
    dn = (((1,), (1,)), ((), ()))
    acc = jnp.broadcast_to(b_ref[...], (BM, FIN)).astype(jnp.float32)
    for p in range(NP):
        a = jnp.concatenate([g_ref[2 * p], g_ref[2 * p + 1]], axis=-1)
        acc = acc + lax.dot_general(
            a.astype(jnp.bfloat16), w_ref[:, 2 * D * p:2 * D * (p + 1)],
            dn, preferred_element_type=jnp.float32)
    o_ref[...] = acc


def _matmul_chunk(c, g, wb, bias, y_prev):
    g_spec = pl.BlockSpec((NUM_ATTR, BM, D), lambda i: (0, i, 0))
    w_spec = pl.BlockSpec((FIN, FIN), lambda i: (0, 0))
    b_spec = pl.BlockSpec((1, FIN), lambda i: (0, 0))
    out_spec = pl.BlockSpec((BM, FIN), lambda i, c=c: (c * CBLK + i, 0))
    out_shape = jax.ShapeDtypeStruct((N, FIN), jnp.float32)

    if y_prev is None:
        return pl.pallas_call(
            _mm_body,
            grid=(CBLK,),
            in_specs=[g_spec, w_spec, b_spec],
            out_specs=out_spec,
            out_shape=out_shape,
        )(g, wb, bias)

    def body(g_ref, w_ref, b_ref, y_ref, o_ref):
        _mm_body(g_ref, w_ref, b_ref, o_ref)

    return pl.pallas_call(
        body,
        grid=(CBLK,),
        in_specs=[g_spec, w_spec, b_spec,
                  pl.BlockSpec(memory_space=pltpu.MemorySpace.HBM)],
        out_specs=out_spec,
        out_shape=out_shape,
        input_output_aliases={3: 0},
    )(g, wb, bias, y_prev)


def kernel(x, tables, W, b):
    xt = x.transpose(2, 1, 0).reshape(NUM_ATTR, N)
    idx = xt + jnp.arange(NUM_ATTR, dtype=jnp.int32)[:, None] * VOCAB
    tab = tables.reshape(NUM_ATTR * VOCAB, D)
    wb = _cast_bf16(W)
    bias = b.reshape(1, FIN)

    gs = []
    for c in range(CHUNKS):
        idx_c = (idx[:, c * ROWS_C:(c + 1) * ROWS_C]
                 .reshape(NUM_ATTR, NW, NCH, CH).transpose(1, 0, 2, 3))
        gs.append(_gather(idx_c, tab))

    y = None
    for c in range(CHUNKS):
        y = _matmul_chunk(c, gs[c], wb, bias, y)

    return y.reshape(OBJ, BATCH, FIN).transpose(1, 0, 2)


# speedup vs baseline: 3.6696x; 1.0047x over previous
import functools

import jax
import jax.numpy as jnp
from jax import lax
from jax.experimental import pallas as pl
from jax.experimental.pallas import tpu as pltpu
from jax.experimental.pallas import tpu_sc as plsc

NUM_ATTR = 26
VOCAB = 1000
D = 128
BATCH = 1024
OBJ = 20
FIN = NUM_ATTR * D
N = BATCH * OBJ

NC = 2
NS = 16
NW = NC * NS
CH = 128

CHUNKS = 5
ROWS_C = N // CHUNKS
ROWS_WC = ROWS_C // NW
NCH = ROWS_WC // CH

NP = NUM_ATTR // 2
BM = 256
CBLK = ROWS_C // BM


def _gather(idx, tab):
    mesh = plsc.VectorSubcoreMesh(core_axis_name="c", subcore_axis_name="s")

    @functools.partial(
        pl.kernel,
        out_type=jax.ShapeDtypeStruct((NUM_ATTR, ROWS_C, D), jnp.float32),
        mesh=mesh,
        scratch_types=[
            pltpu.VMEM((NUM_ATTR, CH), jnp.int32),
            pltpu.VMEM((ROWS_WC, D), jnp.float32),
            pltpu.VMEM((ROWS_WC, D), jnp.float32),
            pltpu.SemaphoreType.DMA,
            pltpu.SemaphoreType.DMA,
        ],
    )
    def sc_kernel(idx_ref, tab_ref, out_ref, idx_v, rows0, rows1, semg, semw):
        wid = lax.axis_index("s") * NC + lax.axis_index("c")
        base = wid * ROWS_WC

        def drain_writes():
            pltpu.make_async_copy(tab_ref.at[pl.ds(0, ROWS_WC)], rows0,
                                  semw).wait()
            pltpu.make_async_copy(tab_ref.at[pl.ds(0, ROWS_WC)], rows1,
                                  semw).wait()

        pltpu.sync_copy(idx_ref.at[wid], idx_v)

        def body(t, carry):
            @pl.when(t > 0)
            def _():
                drain_writes()

            g0 = pltpu.async_copy(tab_ref.at[idx_v.at[2 * t]], rows0, semg)
            g1 = pltpu.async_copy(tab_ref.at[idx_v.at[2 * t + 1]], rows1, semg)
            g0.wait()
            pltpu.async_copy(rows0, out_ref.at[2 * t, pl.ds(base, ROWS_WC)],
                             semw)
            g1.wait()
            pltpu.async_copy(rows1, out_ref.at[2 * t + 1, pl.ds(base, ROWS_WC)],
                             semw)
            return carry

        lax.fori_loop(0, NUM_ATTR // 2, body, 0)
        drain_writes()

    return sc_kernel(idx, tab)


def _cast_bf16(w):
    def body(w_ref, o_ref):
        o_ref[...] = w_ref[...].astype(jnp.bfloat16)

    return pl.pallas_call(
        body,
        grid=(NUM_ATTR,),
        in_specs=[pl.BlockSpec((D, FIN), lambda i: (i, 0))],
        out_specs=pl.BlockSpec((D, FIN), lambda i: (i, 0)),
        out_shape=jax.ShapeDtypeStruct((FIN, FIN), jnp.bfloat16),
    )(w)


def _mm_body(g_ref, w_ref, b_ref, o_ref):
    dn = (((1,), (1,)), ((), ()))
    acc = jnp.broadcast_to(b_ref[...], (BM, FIN)).astype(jnp.float32)
    for p in range(NP):
        a = jnp.concatenate([g_ref[2 * p], g_ref[2 * p + 1]], axis=-1)
        acc = acc + lax.dot_general(
            a.astype(jnp.bfloat16), w_ref[:, 2 * D * p:2 * D * (p + 1)],
            dn, preferred_element_type=jnp.float32)
    o_ref[...] = acc


def _matmul_chunk(c, g, wb, bias, y_prev):
    g_spec = pl.BlockSpec((NUM_ATTR, BM, D), lambda i: (0, i, 0))
    w_spec = pl.BlockSpec((FIN, FIN), lambda i: (0, 0))
    b_spec = pl.BlockSpec((1, FIN), lambda i: (0, 0))
    out_spec = pl.BlockSpec((BM, FIN), lambda i, c=c: (c * CBLK + i, 0))
    out_shape = jax.ShapeDtypeStruct((N, FIN), jnp.float32)

    if y_prev is None:
        return pl.pallas_call(
            _mm_body,
            grid=(CBLK,),
            in_specs=[g_spec, w_spec, b_spec],
            out_specs=out_spec,
            out_shape=out_shape,
        )(g, wb, bias)

    def body(g_ref, w_ref, b_ref, y_ref, o_ref):
        _mm_body(g_ref, w_ref, b_ref, o_ref)

    return pl.pallas_call(
        body,
        grid=(CBLK,),
        in_specs=[g_spec, w_spec, b_spec,
                  pl.BlockSpec(memory_space=pltpu.MemorySpace.HBM)],
        out_specs=out_spec,
        out_shape=out_shape,
        input_output_aliases={3: 0},
    )(g, wb, bias, y_prev)


def kernel(x, tables, W, b):
    xt = x.transpose(2, 1, 0).reshape(NUM_ATTR, N)
    idx = xt + jnp.arange(NUM_ATTR, dtype=jnp.int32)[:, None] * VOCAB
    tab = tables.reshape(NUM_ATTR * VOCAB, D)
    wb = _cast_bf16(W)
    bias = b.reshape(1, FIN)

    gs = []
    for c in range(CHUNKS):
        idx_c = (idx[:, c * ROWS_C:(c + 1) * ROWS_C]
                 .reshape(NUM_ATTR, NW, CH).transpose(1, 0, 2))
        gs.append(_gather(idx_c, tab))

    y = None
    for c in range(CHUNKS):
        y = _matmul_chunk(c, gs[c], wb, bias, y)

    return y.reshape(OBJ, BATCH, FIN).transpose(1, 0, 2)


# speedup vs baseline: 3.6698x; 1.0000x over previous
import functools

import jax
import jax.numpy as jnp
from jax import lax
from jax.experimental import pallas as pl
from jax.experimental.pallas import tpu as pltpu
from jax.experimental.pallas import tpu_sc as plsc

NUM_ATTR = 26
VOCAB = 1000
D = 128
BATCH = 1024
OBJ = 20
FIN = NUM_ATTR * D
N = BATCH * OBJ

NC = 2
NS = 16
NW = NC * NS
CH = 128

CHUNKS = 5
ROWS_C = N // CHUNKS
ROWS_WC = ROWS_C // NW
NCH = ROWS_WC // CH

NP = NUM_ATTR // 2
BM = 512
CBLK = ROWS_C // BM


def _gather(idx, tab):
    mesh = plsc.VectorSubcoreMesh(core_axis_name="c", subcore_axis_name="s")

    @functools.partial(
        pl.kernel,
        out_type=jax.ShapeDtypeStruct((NUM_ATTR, ROWS_C, D), jnp.float32),
        mesh=mesh,
        scratch_types=[
            pltpu.VMEM((NUM_ATTR, CH), jnp.int32),
            pltpu.VMEM((ROWS_WC, D), jnp.float32),
            pltpu.VMEM((ROWS_WC, D), jnp.float32),
            pltpu.SemaphoreType.DMA,
            pltpu.SemaphoreType.DMA,
        ],
    )
    def sc_kernel(idx_ref, tab_ref, out_ref, idx_v, rows0, rows1, semg, semw):
        wid = lax.axis_index("s") * NC + lax.axis_index("c")
        base = wid * ROWS_WC

        def drain_writes():
            pltpu.make_async_copy(tab_ref.at[pl.ds(0, ROWS_WC)], rows0,
                                  semw).wait()
            pltpu.make_async_copy(tab_ref.at[pl.ds(0, ROWS_WC)], rows1,
                                  semw).wait()

        pltpu.sync_copy(idx_ref.at[wid], idx_v)

        def body(t, carry):
            @pl.when(t > 0)
            def _():
                drain_writes()

            g0 = pltpu.async_copy(tab_ref.at[idx_v.at[2 * t]], rows0, semg)
            g1 = pltpu.async_copy(tab_ref.at[idx_v.at[2 * t + 1]], rows1, semg)
            g0.wait()
            pltpu.async_copy(rows0, out_ref.at[2 * t, pl.ds(base, ROWS_WC)],
                             semw)
            g1.wait()
            pltpu.async_copy(rows1, out_ref.at[2 * t + 1, pl.ds(base, ROWS_WC)],
                             semw)
            return carry

        lax.fori_loop(0, NUM_ATTR // 2, body, 0)
        drain_writes()

    return sc_kernel(idx, tab)


def _cast_bf16(w):
    def body(w_ref, o_ref):
        o_ref[...] = w_ref[...].astype(jnp.bfloat16)

    return pl.pallas_call(
        body,
        grid=(NUM_ATTR,),
        in_specs=[pl.BlockSpec((D, FIN), lambda i: (i, 0))],
        out_specs=pl.BlockSpec((D, FIN), lambda i: (i, 0)),
        out_shape=jax.ShapeDtypeStruct((FIN, FIN), jnp.bfloat16),
    )(w)


def _mm_body(g_ref, w_ref, b_ref, o_ref):
    dn = (((1,), (1,)), ((), ()))
    acc = jnp.broadcast_to(b_ref[...], (BM, FIN)).astype(jnp.float32)
    for p in range(NP):
        a = jnp.concatenate([g_ref[2 * p], g_ref[2 * p + 1]], axis=-1)
        acc = acc + lax.dot_general(
            a.astype(jnp.bfloat16), w_ref[:, 2 * D * p:2 * D * (p + 1)],
            dn, preferred_element_type=jnp.float32)
    o_ref[...] = acc


def _matmul_chunk(c, g, wb, bias, y_prev):
    g_spec = pl.BlockSpec((NUM_ATTR, BM, D), lambda i: (0, i, 0))
    w_spec = pl.BlockSpec((FIN, FIN), lambda i: (0, 0))
    b_spec = pl.BlockSpec((1, FIN), lambda i: (0, 0))
    out_spec = pl.BlockSpec((BM, FIN), lambda i, c=c: (c * CBLK + i, 0))
    out_shape = jax.ShapeDtypeStruct((N, FIN), jnp.float32)

    if y_prev is None:
        return pl.pallas_call(
            _mm_body,
            grid=(CBLK,),
            in_specs=[g_spec, w_spec, b_spec],
            out_specs=out_spec,
            out_shape=out_shape,
        )(g, wb, bias)

    def body(g_ref, w_ref, b_ref, y_ref, o_ref):
        _mm_body(g_ref, w_ref, b_ref, o_ref)

    return pl.pallas_call(
        body,
        grid=(CBLK,),
        in_specs=[g_spec, w_spec, b_spec,
                  pl.BlockSpec(memory_space=pltpu.MemorySpace.HBM)],
        out_specs=out_spec,
        out_shape=out_shape,
        input_output_aliases={3: 0},
    )(g, wb, bias, y_prev)


def kernel(x, tables, W, b):
    xt = x.transpose(2, 1, 0).reshape(NUM_ATTR, N)
    idx = xt + jnp.arange(NUM_ATTR, dtype=jnp.int32)[:, None] * VOCAB
    tab = tables.reshape(NUM_ATTR * VOCAB, D)
    wb = _cast_bf16(W)
    bias = b.reshape(1, FIN)

    gs = []
    for c in range(CHUNKS):
        idx_c = (idx[:, c * ROWS_C:(c + 1) * ROWS_C]
                 .reshape(NUM_ATTR, NW, CH).transpose(1, 0, 2))
        gs.append(_gather(idx_c, tab))

    y = None
    for c in range(CHUNKS):
        y = _matmul_chunk(c, gs[c], wb, bias, y)

    return y.reshape(OBJ, BATCH, FIN).transpose(1, 0, 2)
